# baseline (speedup 1.0000x reference)
import jax
import jax.numpy as jnp
from jax import lax
from jax.experimental import pallas as pl
from jax.experimental.pallas import tpu as pltpu

NUM_CLASSES = 1000
CPAD = 1024
FEAT = 2048
BATCH = 16384
SIGMA = 0.2

BBLK = 512
NB = BATCH // BBLK
EBLK = 128
EPI_STEPS = CPAD // EBLK


def _cma_update(rgb_feats, ir_feats, rgb_lab3d, ir_lab3d,
                vis_memory, ir_memory):
  f32 = jnp.float32

  def body(rgb_ref, ir_ref, rlab_ref, ilab_ref, vm_ref, im_ref, out_ref,
           vcnt, icnt):
    i = pl.program_id(0)

    @pl.when(i == 0)
    def _():
      out_ref[...] = jnp.zeros_like(out_ref)
      vcnt[...] = jnp.zeros_like(vcnt)
      icnt[...] = jnp.zeros_like(icnt)

    @pl.when(i < NB)
    def _():
      classes = lax.broadcasted_iota(jnp.int32, (BBLK, CPAD), 1)
      for m, (cnt, lab_ref, f_ref) in enumerate(((vcnt, rlab_ref, rgb_ref),
                                                 (icnt, ilab_ref, ir_ref))):
        onehot = (lab_ref[0, 0, :][:, None] == classes).astype(jnp.bfloat16)
        feats = f_ref[...].astype(jnp.bfloat16)
        out_ref[m] += lax.dot_general(
            onehot, feats, (((0,), (0,)), ((), ())),
            preferred_element_type=f32)
        cnt[...] += jnp.sum(onehot.astype(f32), axis=0, keepdims=True)

    @pl.when(i >= NB)
    def _():
      j = i - NB
      r0 = j * EBLK
      for m, (cnt, mem_ref) in enumerate(((vcnt, vm_ref), (icnt, im_ref))):
        c = cnt[0, pl.ds(r0, EBLK)][:, None]
        mem = mem_ref[...]
        mean = out_ref[m, pl.ds(r0, EBLK), :] / jnp.maximum(c, 1.0)
        upd = (1.0 - SIGMA) * mem + SIGMA * mean
        out_ref[m, pl.ds(r0, EBLK), :] = jnp.where(c > 0.0, upd, mem)

  feat_spec = pl.BlockSpec((BBLK, FEAT),
                           lambda i: (jnp.minimum(i, NB - 1), 0))
  lab_spec = pl.BlockSpec((1, 1, BBLK),
                          lambda i: (jnp.minimum(i, NB - 1), 0, 0))
  mem_spec = pl.BlockSpec(
      (EBLK, FEAT),
      lambda i: (jnp.maximum(i - NB, 0), 0))
  return pl.pallas_call(
      body,
      grid=(NB + EPI_STEPS,),
      in_specs=[feat_spec, feat_spec, lab_spec, lab_spec, mem_spec, mem_spec],
      out_specs=pl.BlockSpec((2, CPAD, FEAT), lambda i: (0, 0, 0)),
      out_shape=jax.ShapeDtypeStruct((2, CPAD, FEAT), f32),
      scratch_shapes=[
          pltpu.VMEM((1, CPAD), f32),
          pltpu.VMEM((1, CPAD), f32),
      ],
  )(rgb_feats, ir_feats, rgb_lab3d, ir_lab3d, vis_memory, ir_memory)


def kernel(rgb_feats, ir_feats, rgb_labels, ir_labels, vis_memory, ir_memory):
  rgb_lab3d = rgb_labels.astype(jnp.int32).reshape(NB, 1, BBLK)
  ir_lab3d = ir_labels.astype(jnp.int32).reshape(NB, 1, BBLK)
  out = _cma_update(rgb_feats, ir_feats, rgb_lab3d, ir_lab3d,
                    vis_memory, ir_memory)
  return out[:, :NUM_CLASSES, :]
